# Initial kernel scaffold; baseline (speedup 1.0000x reference)
import jax
import jax.numpy as jnp
from jax.experimental import pallas as pl


def kernel(x, H, adj, nhid, W_i, a1_i, a2_i, b_i, Wres_i, bres_i, W_a, a1_a, a2_a, b_a, Wres_a, bres_a):
    raise NotImplementedError("write your pallas kernel here")


# speedup vs baseline: 1.8845x; 1.8845x over previous
import functools

import jax
import jax.numpy as jnp
from jax.experimental import pallas as pl


def _proj_body(x_ref, w_ref, b_ref, o_ref):
    o_ref[0] = (
        jnp.dot(x_ref[0], w_ref[...], preferred_element_type=jnp.float32)
        + b_ref[...]
    )


def _attn_body(proj_ref, adj_ref, hm_ref, par_ref, allhf_ref, he_ref, ind_ref,
               *, HD, E, RB, SP):
    rb = pl.program_id(1)

    V_i = proj_ref[0, :, 0:HD]
    V_a = proj_ref[0, :, 2 * HD:3 * HD]
    hm = hm_ref[...]

    a1_i = par_ref[0:1, :]
    a2_i = par_ref[1:2, :]
    b_i = par_ref[2:3, :]
    a1_a = par_ref[3:4, :]
    a2_a = par_ref[4:5, :]
    b_a = par_ref[5:6, :]

    row0 = rb * RB
    res_i = proj_ref[0, pl.ds(row0, RB), HD:2 * HD]
    res_a = proj_ref[0, pl.ds(row0, RB), 3 * HD:4 * HD]
    Vr_i = proj_ref[0, pl.ds(row0, RB), 0:HD]
    Vr_a = proj_ref[0, pl.ds(row0, RB), 2 * HD:3 * HD]

    def weights(V_full, Vr, a1, a2):
        f2 = jax.lax.dot_general(a2, V_full, (((1,), (1,)), ((), ())),
                                 preferred_element_type=jnp.float32)
        m2 = jnp.max(f2)
        t = f2 - m2
        wp_c = jnp.exp(t)
        wn_c = jnp.exp(0.2 * t)
        f1 = jax.lax.dot_general(Vr, a1, (((1,), (1,)), ((), ())),
                                 preferred_element_type=jnp.float32)
        u = f1 + m2
        m = jnp.where(u > 0, u, 0.2 * u)
        wp_r = jnp.exp(u - m)
        wn_r = jnp.exp(0.2 * u - m)
        s = f1 + f2
        return jnp.where(s > 0, wp_r * wp_c, wn_r * wn_c)

    w = weights(V_i, Vr_i, a1_i, a2_i)
    T = jnp.concatenate(
        [hm[:, e:e + 1] * V_i for e in range(E)] + [hm], axis=1
    )
    acc = jnp.dot(w, T, preferred_element_type=jnp.float32)

    hmr = hm_ref[pl.ds(row0, RB), :]
    he_parts = []
    for e in range(E):
        num = acc[:, e * HD:(e + 1) * HD]
        den = acc[:, E * HD + e:E * HD + e + 1]
        z = num / den + b_i + res_i
        after = jnp.where(z > 0, z, jnp.exp(z) - 1.0)
        meme = hmr[:, e:e + 1]
        allhf_ref[e, 0] = meme * after
        he_parts.append(jnp.max(jnp.where(meme > 0, after, -1e30),
                                axis=0, keepdims=True))
    he_new = jnp.concatenate(he_parts, axis=0)

    @pl.when(rb == 0)
    def _():
        he_ref[0] = he_new

    @pl.when(rb > 0)
    def _():
        he_ref[0] = jnp.maximum(he_ref[0], he_new)

    w_a = weights(V_a, Vr_a, a1_a, a2_a) * adj_ref[...]
    Vaug = jnp.concatenate(
        [V_a, jnp.ones((V_a.shape[0], 1), jnp.float32)], axis=1
    )
    acc_a = jnp.dot(w_a, Vaug, preferred_element_type=jnp.float32)
    z = acc_a[:, 0:HD] / acc_a[:, HD:HD + 1] + b_a + res_a
    ind_ref[0] = jnp.where(z > 0, z, jnp.exp(z) - 1.0)


def kernel(x, H, adj, nhid, W_i, a1_i, a2_i, b_i, Wres_i, bres_i,
           W_a, a1_a, a2_a, b_a, Wres_a, bres_a):
    B, S, F = x.shape
    HD = W_i.shape[1]
    E = H.shape[1]
    SP = ((S + 255) // 256) * 256
    RB = 400
    NRB = S // RB

    x_p = jnp.pad(x, ((0, 0), (0, SP - S), (0, 0)))
    hm = jnp.pad((H != 0).astype(jnp.float32), ((0, SP - S), (0, 0)))
    adjf = jnp.pad(adj.astype(jnp.float32), ((0, 0), (0, SP - S)))
    Wcat = jnp.concatenate([W_i, Wres_i, W_a, Wres_a], axis=1)
    zeros = jnp.zeros((HD,), jnp.float32)
    bias_cat = jnp.concatenate([zeros, bres_i, zeros, bres_a])[None, :]
    params = jnp.stack([a1_i[:, 0], a2_i[:, 0], b_i,
                        a1_a[:, 0], a2_a[:, 0], b_a, zeros, zeros])

    PB = 256
    proj = pl.pallas_call(
        _proj_body,
        grid=(B, SP // PB),
        in_specs=[
            pl.BlockSpec((1, PB, F), lambda b, r: (b, r, 0)),
            pl.BlockSpec((F, 4 * HD), lambda b, r: (0, 0)),
            pl.BlockSpec((1, 4 * HD), lambda b, r: (0, 0)),
        ],
        out_specs=pl.BlockSpec((1, PB, 4 * HD), lambda b, r: (b, r, 0)),
        out_shape=jax.ShapeDtypeStruct((B, SP, 4 * HD), jnp.float32),
    )(x_p, Wcat, bias_cat)

    allhf, he, ind = pl.pallas_call(
        functools.partial(_attn_body, HD=HD, E=E, RB=RB, SP=SP),
        grid=(B, NRB),
        in_specs=[
            pl.BlockSpec((1, SP, 4 * HD), lambda b, r: (b, 0, 0)),
            pl.BlockSpec((RB, SP), lambda b, r: (r, 0)),
            pl.BlockSpec((SP, E), lambda b, r: (0, 0)),
            pl.BlockSpec((8, HD), lambda b, r: (0, 0)),
        ],
        out_specs=[
            pl.BlockSpec((E, 1, RB, HD), lambda b, r: (0, b, r, 0)),
            pl.BlockSpec((1, E, HD), lambda b, r: (b, 0, 0)),
            pl.BlockSpec((1, RB, HD), lambda b, r: (b, r, 0)),
        ],
        out_shape=[
            jax.ShapeDtypeStruct((E, B, S, HD), jnp.float32),
            jax.ShapeDtypeStruct((B, E, HD), jnp.float32),
            jax.ShapeDtypeStruct((B, S, HD), jnp.float32),
        ],
    )(proj, adjf, hm, params)

    return (allhf, he, ind)


# speedup vs baseline: 2.1249x; 1.1276x over previous
import functools

import jax
import jax.numpy as jnp
from jax.experimental import pallas as pl


def _proj_body(x_ref, w_ref, b_ref, o_ref):
    o_ref[0] = (
        jnp.dot(x_ref[0], w_ref[...], preferred_element_type=jnp.float32)
        + b_ref[...]
    )


def _attn_body(proj_ref, adj_ref, hm_ref, par_ref, allhf_ref, he_ref, ind_ref,
               *, HD, E, RB, SP):
    rb = pl.program_id(1)

    V_i = proj_ref[0, :, 0:HD]
    V_a = proj_ref[0, :, 2 * HD:3 * HD]
    hm = hm_ref[...]

    a1_i = par_ref[0:1, :]
    a2_i = par_ref[1:2, :]
    b_i = par_ref[2:3, :]
    a1_a = par_ref[3:4, :]
    a2_a = par_ref[4:5, :]
    b_a = par_ref[5:6, :]

    row0 = rb * RB
    res_i = proj_ref[0, pl.ds(row0, RB), HD:2 * HD]
    res_a = proj_ref[0, pl.ds(row0, RB), 3 * HD:4 * HD]
    Vr_i = proj_ref[0, pl.ds(row0, RB), 0:HD]
    Vr_a = proj_ref[0, pl.ds(row0, RB), 2 * HD:3 * HD]

    def weights(V_full, Vr, a1, a2):
        f2 = jax.lax.dot_general(a2, V_full, (((1,), (1,)), ((), ())),
                                 preferred_element_type=jnp.float32)
        m2 = jnp.max(f2)
        t = f2 - m2
        wp_c = jnp.exp(t)
        wn_c = jnp.exp(0.2 * t)
        f1 = jax.lax.dot_general(Vr, a1, (((1,), (1,)), ((), ())),
                                 preferred_element_type=jnp.float32)
        u = f1 + m2
        m = jnp.where(u > 0, u, 0.2 * u)
        wp_r = jnp.exp(u - m)
        wn_r = jnp.exp(0.2 * u - m)
        s = f1 + f2
        return jnp.where(s > 0, wp_r * wp_c, wn_r * wn_c)

    w = weights(V_i, Vr_i, a1_i, a2_i).astype(jnp.bfloat16)
    Vb = V_i.astype(jnp.bfloat16)
    hmb = hm.astype(jnp.bfloat16)
    T = jnp.concatenate(
        [hmb[:, e:e + 1] * Vb for e in range(E)] + [hmb], axis=1
    )
    acc = jnp.dot(w, T, preferred_element_type=jnp.float32)

    hmr = hm_ref[pl.ds(row0, RB), :]
    he_parts = []
    for e in range(E):
        num = acc[:, e * HD:(e + 1) * HD]
        den = acc[:, E * HD + e:E * HD + e + 1]
        z = num / den + b_i + res_i
        after = jnp.where(z > 0, z, jnp.exp(z) - 1.0)
        meme = hmr[:, e:e + 1]
        allhf_ref[e, 0] = meme * after
        he_parts.append(jnp.max(jnp.where(meme > 0, after, -1e30),
                                axis=0, keepdims=True))
    he_new = jnp.concatenate(he_parts, axis=0)

    @pl.when(rb == 0)
    def _():
        he_ref[0] = he_new

    @pl.when(rb > 0)
    def _():
        he_ref[0] = jnp.maximum(he_ref[0], he_new)

    w_a = (weights(V_a, Vr_a, a1_a, a2_a) * adj_ref[...]).astype(jnp.bfloat16)
    Vaug = jnp.concatenate(
        [V_a.astype(jnp.bfloat16),
         jnp.ones((V_a.shape[0], 1), jnp.bfloat16)], axis=1
    )
    acc_a = jnp.dot(w_a, Vaug, preferred_element_type=jnp.float32)
    z = acc_a[:, 0:HD] / acc_a[:, HD:HD + 1] + b_a + res_a
    ind_ref[0] = jnp.where(z > 0, z, jnp.exp(z) - 1.0)


def kernel(x, H, adj, nhid, W_i, a1_i, a2_i, b_i, Wres_i, bres_i,
           W_a, a1_a, a2_a, b_a, Wres_a, bres_a):
    B, S, F = x.shape
    HD = W_i.shape[1]
    E = H.shape[1]
    SP = ((S + 255) // 256) * 256
    RB = 400
    NRB = S // RB

    x_p = jnp.pad(x, ((0, 0), (0, SP - S), (0, 0)))
    hm = jnp.pad((H != 0).astype(jnp.float32), ((0, SP - S), (0, 0)))
    adjf = jnp.pad(adj.astype(jnp.float32), ((0, 0), (0, SP - S)))
    Wcat = jnp.concatenate([W_i, Wres_i, W_a, Wres_a], axis=1)
    zeros = jnp.zeros((HD,), jnp.float32)
    bias_cat = jnp.concatenate([zeros, bres_i, zeros, bres_a])[None, :]
    params = jnp.stack([a1_i[:, 0], a2_i[:, 0], b_i,
                        a1_a[:, 0], a2_a[:, 0], b_a, zeros, zeros])

    PB = 256
    proj = pl.pallas_call(
        _proj_body,
        grid=(B, SP // PB),
        in_specs=[
            pl.BlockSpec((1, PB, F), lambda b, r: (b, r, 0)),
            pl.BlockSpec((F, 4 * HD), lambda b, r: (0, 0)),
            pl.BlockSpec((1, 4 * HD), lambda b, r: (0, 0)),
        ],
        out_specs=pl.BlockSpec((1, PB, 4 * HD), lambda b, r: (b, r, 0)),
        out_shape=jax.ShapeDtypeStruct((B, SP, 4 * HD), jnp.float32),
    )(x_p, Wcat, bias_cat)

    allhf, he, ind = pl.pallas_call(
        functools.partial(_attn_body, HD=HD, E=E, RB=RB, SP=SP),
        grid=(B, NRB),
        in_specs=[
            pl.BlockSpec((1, SP, 4 * HD), lambda b, r: (b, 0, 0)),
            pl.BlockSpec((RB, SP), lambda b, r: (r, 0)),
            pl.BlockSpec((SP, E), lambda b, r: (0, 0)),
            pl.BlockSpec((8, HD), lambda b, r: (0, 0)),
        ],
        out_specs=[
            pl.BlockSpec((E, 1, RB, HD), lambda b, r: (0, b, r, 0)),
            pl.BlockSpec((1, E, HD), lambda b, r: (b, 0, 0)),
            pl.BlockSpec((1, RB, HD), lambda b, r: (b, r, 0)),
        ],
        out_shape=[
            jax.ShapeDtypeStruct((E, B, S, HD), jnp.float32),
            jax.ShapeDtypeStruct((B, E, HD), jnp.float32),
            jax.ShapeDtypeStruct((B, S, HD), jnp.float32),
        ],
    )(proj, adjf, hm, params)

    return (allhf, he, ind)


# speedup vs baseline: 2.5096x; 1.1811x over previous
import functools

import jax
import jax.numpy as jnp
from jax.experimental import pallas as pl


def _proj_body(x_ref, wi_ref, wri_ref, wa_ref, wra_ref, bri_ref, bra_ref,
               hm_ref, spr_ref, vi_ref, ri_ref, va_ref, ra_ref, t_ref,
               vab_ref):
    x = x_ref[0]
    vi = jnp.dot(x, wi_ref[...], preferred_element_type=jnp.float32)
    va = jnp.dot(x, wa_ref[...], preferred_element_type=jnp.float32)
    vi_ref[0] = vi
    va_ref[0] = va
    ri_ref[0] = (jnp.dot(x, wri_ref[...], preferred_element_type=jnp.float32)
                 + bri_ref[...])
    ra_ref[0] = (jnp.dot(x, wra_ref[...], preferred_element_type=jnp.float32)
                 + bra_ref[...])
    hmb = hm_ref[...].astype(jnp.bfloat16)
    mb = jnp.dot(hmb, spr_ref[...].astype(jnp.bfloat16),
                 preferred_element_type=jnp.float32
                 ).astype(jnp.bfloat16)
    vib = vi.astype(jnp.bfloat16)
    E = hm_ref.shape[1]
    HD = vi.shape[1]
    t_ref[0] = jnp.concatenate(
        [mb[:, e * HD:(e + 1) * HD] * vib for e in range(E)] + [hmb], axis=1)
    PB = x.shape[0]
    vab_ref[0] = jnp.concatenate(
        [va.astype(jnp.bfloat16),
         jnp.ones((PB, 1), jnp.bfloat16),
         jnp.zeros((PB, 7), jnp.bfloat16)], axis=1)


def _attn_body(vi_ref, va_ref, ri_ref, ra_ref, t_ref, vab_ref, adj_ref,
               hm_ref, spr_ref, par_ref, allhf_ref, he_ref, ind_ref,
               *, HD, E, RB):
    rb = pl.program_id(1)
    row0 = rb * RB

    a1_i = par_ref[0:1, :]
    a2_i = par_ref[1:2, :]
    b_i = par_ref[2:3, :]
    a1_a = par_ref[3:4, :]
    a2_a = par_ref[4:5, :]
    b_a = par_ref[5:6, :]

    def weights(v_ref, a1, a2):
        v = v_ref[0]
        f2 = jax.lax.dot_general(a2, v, (((1,), (1,)), ((), ())),
                                 preferred_element_type=jnp.float32)
        m2 = jnp.max(f2)
        t = f2 - m2
        wp_c = jnp.exp(t).astype(jnp.bfloat16)
        wn_c = jnp.exp(0.2 * t).astype(jnp.bfloat16)
        vr = v_ref[0, pl.ds(row0, RB), :]
        f1 = jax.lax.dot_general(vr, a1, (((1,), (1,)), ((), ())),
                                 preferred_element_type=jnp.float32)
        u = f1 + m2
        m = jnp.where(u > 0, u, 0.2 * u)
        wp_r = jnp.exp(u - m).astype(jnp.bfloat16)
        wn_r = jnp.exp(0.2 * u - m).astype(jnp.bfloat16)
        s = f1.astype(jnp.bfloat16) + f2.astype(jnp.bfloat16)
        return jnp.where(s > 0, wp_r * wp_c, wn_r * wn_c)

    w = weights(vi_ref, a1_i, a2_i)
    acc = jnp.dot(w, t_ref[0], preferred_element_type=jnp.float32)
    sprb = spr_ref[...].astype(jnp.bfloat16)
    rden = 1.0 / acc[:, E * HD:E * HD + E]
    rdenb = jnp.dot(rden, spr_ref[...],
                    preferred_element_type=jnp.float32)
    maskb = jnp.dot(hm_ref[...].astype(jnp.bfloat16), sprb,
                    preferred_element_type=jnp.float32)

    he_parts = []
    for e in range(E):
        sl = slice(e * HD, (e + 1) * HD)
        z = acc[:, sl] * rdenb[:, sl] + b_i + ri_ref[0]
        after = jnp.where(z > 0, z, jnp.exp(z) - 1.0)
        allhf_ref[e, 0] = maskb[:, sl] * after
        he_parts.append(jnp.max(jnp.where(maskb[:, sl] > 0, after, -1e30),
                                axis=0, keepdims=True))
    he_new = jnp.concatenate(he_parts, axis=0)

    @pl.when(rb == 0)
    def _():
        he_ref[0] = he_new

    @pl.when(rb > 0)
    def _():
        he_ref[0] = jnp.maximum(he_ref[0], he_new)

    w_a = weights(va_ref, a1_a, a2_a) * adj_ref[...]
    acc_a = jnp.dot(w_a, vab_ref[0], preferred_element_type=jnp.float32)
    rden_a = 1.0 / acc_a[:, HD:HD + 1]
    ones_row = jnp.ones((1, HD), jnp.float32)
    rdab = jnp.dot(rden_a, ones_row, preferred_element_type=jnp.float32)
    z = acc_a[:, 0:HD] * rdab + b_a + ra_ref[0]
    ind_ref[0] = jnp.where(z > 0, z, jnp.exp(z) - 1.0)


def kernel(x, H, adj, nhid, W_i, a1_i, a2_i, b_i, Wres_i, bres_i,
           W_a, a1_a, a2_a, b_a, Wres_a, bres_a):
    B, S, F = x.shape
    HD = W_i.shape[1]
    E = H.shape[1]
    SP = ((S + 255) // 256) * 256
    RB = 400
    NRB = S // RB

    x_p = jnp.pad(x, ((0, 0), (0, SP - S), (0, 0)))
    hm = jnp.pad((H != 0).astype(jnp.float32), ((0, SP - S), (0, 0)))
    adjb = jnp.pad(adj.astype(jnp.bfloat16), ((0, 0), (0, SP - S)))
    zeros = jnp.zeros((HD,), jnp.float32)
    params = jnp.stack([a1_i[:, 0], a2_i[:, 0], b_i,
                        a1_a[:, 0], a2_a[:, 0], b_a, zeros, zeros])
    spread = (jnp.arange(E)[:, None] ==
              (jnp.arange(E * HD) // HD)[None, :]).astype(jnp.float32)

    PB = 256
    vi, ri, va, ra, t, vab = pl.pallas_call(
        _proj_body,
        grid=(B, SP // PB),
        in_specs=[
            pl.BlockSpec((1, PB, F), lambda b, r: (b, r, 0)),
            pl.BlockSpec((F, HD), lambda b, r: (0, 0)),
            pl.BlockSpec((F, HD), lambda b, r: (0, 0)),
            pl.BlockSpec((F, HD), lambda b, r: (0, 0)),
            pl.BlockSpec((F, HD), lambda b, r: (0, 0)),
            pl.BlockSpec((1, HD), lambda b, r: (0, 0)),
            pl.BlockSpec((1, HD), lambda b, r: (0, 0)),
            pl.BlockSpec((PB, E), lambda b, r: (r, 0)),
            pl.BlockSpec((E, E * HD), lambda b, r: (0, 0)),
        ],
        out_specs=[
            pl.BlockSpec((1, PB, HD), lambda b, r: (b, r, 0)),
            pl.BlockSpec((1, PB, HD), lambda b, r: (b, r, 0)),
            pl.BlockSpec((1, PB, HD), lambda b, r: (b, r, 0)),
            pl.BlockSpec((1, PB, HD), lambda b, r: (b, r, 0)),
            pl.BlockSpec((1, PB, E * HD + E), lambda b, r: (b, r, 0)),
            pl.BlockSpec((1, PB, HD + 8), lambda b, r: (b, r, 0)),
        ],
        out_shape=[
            jax.ShapeDtypeStruct((B, SP, HD), jnp.float32),
            jax.ShapeDtypeStruct((B, SP, HD), jnp.float32),
            jax.ShapeDtypeStruct((B, SP, HD), jnp.float32),
            jax.ShapeDtypeStruct((B, SP, HD), jnp.float32),
            jax.ShapeDtypeStruct((B, SP, E * HD + E), jnp.bfloat16),
            jax.ShapeDtypeStruct((B, SP, HD + 8), jnp.bfloat16),
        ],
    )(x_p, W_i, Wres_i, W_a, Wres_a, bres_i[None, :], bres_a[None, :],
      hm, spread)

    allhf, he, ind = pl.pallas_call(
        functools.partial(_attn_body, HD=HD, E=E, RB=RB),
        grid=(B, NRB),
        in_specs=[
            pl.BlockSpec((1, SP, HD), lambda b, r: (b, 0, 0)),
            pl.BlockSpec((1, SP, HD), lambda b, r: (b, 0, 0)),
            pl.BlockSpec((1, RB, HD), lambda b, r: (b, r, 0)),
            pl.BlockSpec((1, RB, HD), lambda b, r: (b, r, 0)),
            pl.BlockSpec((1, SP, E * HD + E), lambda b, r: (b, 0, 0)),
            pl.BlockSpec((1, SP, HD + 8), lambda b, r: (b, 0, 0)),
            pl.BlockSpec((RB, SP), lambda b, r: (r, 0)),
            pl.BlockSpec((RB, E), lambda b, r: (r, 0)),
            pl.BlockSpec((E, E * HD), lambda b, r: (0, 0)),
            pl.BlockSpec((8, HD), lambda b, r: (0, 0)),
        ],
        out_specs=[
            pl.BlockSpec((E, 1, RB, HD), lambda b, r: (0, b, r, 0)),
            pl.BlockSpec((1, E, HD), lambda b, r: (b, 0, 0)),
            pl.BlockSpec((1, RB, HD), lambda b, r: (b, r, 0)),
        ],
        out_shape=[
            jax.ShapeDtypeStruct((E, B, S, HD), jnp.float32),
            jax.ShapeDtypeStruct((B, E, HD), jnp.float32),
            jax.ShapeDtypeStruct((B, S, HD), jnp.float32),
        ],
    )(vi, va, ri, ra, t, vab, adjb, hm[:S], spread, params)

    return (allhf, he, ind)


# speedup vs baseline: 2.8473x; 1.1345x over previous
import functools

import jax
import jax.numpy as jnp
from jax.experimental import pallas as pl
from jax.experimental.pallas import tpu as pltpu


def _proj_body(x_ref, wi_ref, wri_ref, wa_ref, wra_ref, bri_ref, bra_ref,
               hm_ref, spr_ref, vi_ref, ri_ref, va_ref, ra_ref, t_ref,
               vab_ref):
    x = x_ref[0]
    vi = jnp.dot(x, wi_ref[...], preferred_element_type=jnp.float32)
    va = jnp.dot(x, wa_ref[...], preferred_element_type=jnp.float32)
    vi_ref[0] = vi
    va_ref[0] = va
    ri_ref[0] = (jnp.dot(x, wri_ref[...], preferred_element_type=jnp.float32)
                 + bri_ref[...])
    ra_ref[0] = (jnp.dot(x, wra_ref[...], preferred_element_type=jnp.float32)
                 + bra_ref[...])
    hmb = hm_ref[...].astype(jnp.bfloat16)
    mb = jnp.dot(hmb, spr_ref[...].astype(jnp.bfloat16),
                 preferred_element_type=jnp.float32
                 ).astype(jnp.bfloat16)
    vib = vi.astype(jnp.bfloat16)
    E = hm_ref.shape[1]
    HD = vi.shape[1]
    t_ref[0] = jnp.concatenate(
        [mb[:, e * HD:(e + 1) * HD] * vib for e in range(E)] + [hmb], axis=1)
    PB = x.shape[0]
    vab_ref[0] = jnp.concatenate(
        [va.astype(jnp.bfloat16),
         jnp.ones((PB, 1), jnp.bfloat16),
         jnp.zeros((PB, 7), jnp.bfloat16)], axis=1)


def _attn_body(vi_ref, va_ref, ri_ref, ra_ref, t_ref, vab_ref, adj_ref,
               hm_ref, spr_ref, par_ref, allhf_ref, he_ref, ind_ref,
               *, HD, E, RB):
    rb = pl.program_id(1)
    row0 = rb * RB

    a1_i = par_ref[0:1, :]
    a2_i = par_ref[1:2, :]
    b_i = par_ref[2:3, :]
    a1_a = par_ref[3:4, :]
    a2_a = par_ref[4:5, :]
    b_a = par_ref[5:6, :]

    def weights(v_ref, a1, a2):
        v = v_ref[0]
        f2 = jax.lax.dot_general(a2, v, (((1,), (1,)), ((), ())),
                                 preferred_element_type=jnp.float32)
        m2 = jnp.max(f2)
        t = f2 - m2
        wp_c = jnp.exp(t).astype(jnp.bfloat16)
        wn_c = jnp.exp(0.2 * t).astype(jnp.bfloat16)
        vr = v_ref[0, pl.ds(row0, RB), :]
        f1 = jax.lax.dot_general(vr, a1, (((1,), (1,)), ((), ())),
                                 preferred_element_type=jnp.float32)
        u = f1 + m2
        m = jnp.where(u > 0, u, 0.2 * u)
        wp_r = jnp.exp(u - m).astype(jnp.bfloat16)
        wn_r = jnp.exp(0.2 * u - m).astype(jnp.bfloat16)
        s = f1.astype(jnp.bfloat16) + f2.astype(jnp.bfloat16)
        return jnp.where(s > 0, wp_r * wp_c, wn_r * wn_c)

    w = weights(vi_ref, a1_i, a2_i)
    acc = jnp.dot(w, t_ref[0], preferred_element_type=jnp.float32)
    sprb = spr_ref[...].astype(jnp.bfloat16)
    rden = 1.0 / acc[:, E * HD:E * HD + E]
    rdenb = jnp.dot(rden, spr_ref[...],
                    preferred_element_type=jnp.float32)
    maskb = jnp.dot(hm_ref[...].astype(jnp.bfloat16), sprb,
                    preferred_element_type=jnp.float32)

    he_parts = []
    for e in range(E):
        sl = slice(e * HD, (e + 1) * HD)
        z = acc[:, sl] * rdenb[:, sl] + b_i + ri_ref[0]
        after = jnp.where(z > 0, z, jnp.exp(z) - 1.0)
        allhf_ref[e, 0] = maskb[:, sl] * after
        he_parts.append(jnp.max(jnp.where(maskb[:, sl] > 0, after, -1e30),
                                axis=0, keepdims=True))
    he_new = jnp.concatenate(he_parts, axis=0)

    @pl.when(rb == 0)
    def _():
        he_ref[0] = he_new

    @pl.when(rb > 0)
    def _():
        he_ref[0] = jnp.maximum(he_ref[0], he_new)

    w_a = weights(va_ref, a1_a, a2_a) * adj_ref[...]
    acc_a = jnp.dot(w_a, vab_ref[0], preferred_element_type=jnp.float32)
    rden_a = 1.0 / acc_a[:, HD:HD + 1]
    ones_row = jnp.ones((1, HD), jnp.float32)
    rdab = jnp.dot(rden_a, ones_row, preferred_element_type=jnp.float32)
    z = acc_a[:, 0:HD] * rdab + b_a + ra_ref[0]
    ind_ref[0] = jnp.where(z > 0, z, jnp.exp(z) - 1.0)


def kernel(x, H, adj, nhid, W_i, a1_i, a2_i, b_i, Wres_i, bres_i,
           W_a, a1_a, a2_a, b_a, Wres_a, bres_a):
    B, S, F = x.shape
    HD = W_i.shape[1]
    E = H.shape[1]
    SP = ((S + 255) // 256) * 256
    RB = 2000
    NRB = S // RB

    x_p = jnp.pad(x, ((0, 0), (0, SP - S), (0, 0)))
    hm = jnp.pad((H != 0).astype(jnp.float32), ((0, SP - S), (0, 0)))
    adjb = jnp.pad(adj.astype(jnp.bfloat16), ((0, 0), (0, SP - S)))
    zeros = jnp.zeros((HD,), jnp.float32)
    params = jnp.stack([a1_i[:, 0], a2_i[:, 0], b_i,
                        a1_a[:, 0], a2_a[:, 0], b_a, zeros, zeros])
    spread = (jnp.arange(E)[:, None] ==
              (jnp.arange(E * HD) // HD)[None, :]).astype(jnp.float32)

    PB = 256
    vi, ri, va, ra, t, vab = pl.pallas_call(
        _proj_body,
        grid=(B, SP // PB),
        in_specs=[
            pl.BlockSpec((1, PB, F), lambda b, r: (b, r, 0)),
            pl.BlockSpec((F, HD), lambda b, r: (0, 0)),
            pl.BlockSpec((F, HD), lambda b, r: (0, 0)),
            pl.BlockSpec((F, HD), lambda b, r: (0, 0)),
            pl.BlockSpec((F, HD), lambda b, r: (0, 0)),
            pl.BlockSpec((1, HD), lambda b, r: (0, 0)),
            pl.BlockSpec((1, HD), lambda b, r: (0, 0)),
            pl.BlockSpec((PB, E), lambda b, r: (r, 0)),
            pl.BlockSpec((E, E * HD), lambda b, r: (0, 0)),
        ],
        out_specs=[
            pl.BlockSpec((1, PB, HD), lambda b, r: (b, r, 0)),
            pl.BlockSpec((1, PB, HD), lambda b, r: (b, r, 0)),
            pl.BlockSpec((1, PB, HD), lambda b, r: (b, r, 0)),
            pl.BlockSpec((1, PB, HD), lambda b, r: (b, r, 0)),
            pl.BlockSpec((1, PB, E * HD + E), lambda b, r: (b, r, 0)),
            pl.BlockSpec((1, PB, HD + 8), lambda b, r: (b, r, 0)),
        ],
        out_shape=[
            jax.ShapeDtypeStruct((B, SP, HD), jnp.float32),
            jax.ShapeDtypeStruct((B, SP, HD), jnp.float32),
            jax.ShapeDtypeStruct((B, SP, HD), jnp.float32),
            jax.ShapeDtypeStruct((B, SP, HD), jnp.float32),
            jax.ShapeDtypeStruct((B, SP, E * HD + E), jnp.bfloat16),
            jax.ShapeDtypeStruct((B, SP, HD + 8), jnp.bfloat16),
        ],
    )(x_p, W_i, Wres_i, W_a, Wres_a, bres_i[None, :], bres_a[None, :],
      hm, spread)

    allhf, he, ind = pl.pallas_call(
        functools.partial(_attn_body, HD=HD, E=E, RB=RB),
        grid=(B, NRB),
        in_specs=[
            pl.BlockSpec((1, SP, HD), lambda b, r: (b, 0, 0)),
            pl.BlockSpec((1, SP, HD), lambda b, r: (b, 0, 0)),
            pl.BlockSpec((1, RB, HD), lambda b, r: (b, r, 0)),
            pl.BlockSpec((1, RB, HD), lambda b, r: (b, r, 0)),
            pl.BlockSpec((1, SP, E * HD + E), lambda b, r: (b, 0, 0)),
            pl.BlockSpec((1, SP, HD + 8), lambda b, r: (b, 0, 0)),
            pl.BlockSpec((RB, SP), lambda b, r: (r, 0)),
            pl.BlockSpec((RB, E), lambda b, r: (r, 0)),
            pl.BlockSpec((E, E * HD), lambda b, r: (0, 0)),
            pl.BlockSpec((8, HD), lambda b, r: (0, 0)),
        ],
        out_specs=[
            pl.BlockSpec((E, 1, RB, HD), lambda b, r: (0, b, r, 0)),
            pl.BlockSpec((1, E, HD), lambda b, r: (b, 0, 0)),
            pl.BlockSpec((1, RB, HD), lambda b, r: (b, r, 0)),
        ],
        out_shape=[
            jax.ShapeDtypeStruct((E, B, S, HD), jnp.float32),
            jax.ShapeDtypeStruct((B, E, HD), jnp.float32),
            jax.ShapeDtypeStruct((B, S, HD), jnp.float32),
        ],
        compiler_params=pltpu.CompilerParams(
            vmem_limit_bytes=100 * 1024 * 1024),
    )(vi, va, ri, ra, t, vab, adjb, hm[:S], spread, params)

    return (allhf, he, ind)


# speedup vs baseline: 3.7000x; 1.2995x over previous
import functools

import jax
import jax.numpy as jnp
from jax.experimental import pallas as pl
from jax.experimental.pallas import tpu as pltpu


def _body(x_ref, adj_ref, hm_ref, spr_ref, par_ref, wi_ref, wri_ref,
          wa_ref, wra_ref, bri_ref, bra_ref, allhf_ref, he_ref, ind_ref,
          *, HD, E):
    xb = x_ref[0].astype(jnp.bfloat16)
    v_i = jnp.dot(xb, wi_ref[...].astype(jnp.bfloat16),
                  preferred_element_type=jnp.float32)
    v_a = jnp.dot(xb, wa_ref[...].astype(jnp.bfloat16),
                  preferred_element_type=jnp.float32)
    res_i = jnp.dot(xb, wri_ref[...].astype(jnp.bfloat16),
                    preferred_element_type=jnp.float32) + bri_ref[...]
    res_a = jnp.dot(xb, wra_ref[...].astype(jnp.bfloat16),
                    preferred_element_type=jnp.float32) + bra_ref[...]

    a1_i = par_ref[0:1, :]
    a2_i = par_ref[1:2, :]
    b_i = par_ref[2:3, :]
    a1_a = par_ref[3:4, :]
    a2_a = par_ref[4:5, :]
    b_a = par_ref[5:6, :]

    def weights(v, a1, a2):
        f2 = jax.lax.dot_general(a2, v, (((1,), (1,)), ((), ())),
                                 preferred_element_type=jnp.float32)
        m2 = jnp.max(f2)
        t = f2 - m2
        wp_c = jnp.exp(t).astype(jnp.bfloat16)
        wn_c = jnp.exp(0.2 * t).astype(jnp.bfloat16)
        f1 = jax.lax.dot_general(v, a1, (((1,), (1,)), ((), ())),
                                 preferred_element_type=jnp.float32)
        u = f1 + m2
        m = jnp.where(u > 0, u, 0.2 * u)
        wp_r = jnp.exp(u - m).astype(jnp.bfloat16)
        wn_r = jnp.exp(0.2 * u - m).astype(jnp.bfloat16)
        s = f1.astype(jnp.bfloat16) + f2.astype(jnp.bfloat16)
        return jnp.where(s > 0, wp_r * wp_c, wn_r * wn_c)

    hmb = hm_ref[...].astype(jnp.bfloat16)
    sprb = spr_ref[...].astype(jnp.bfloat16)
    mb = jnp.dot(hmb, sprb,
                 preferred_element_type=jnp.float32)
    mbb = mb.astype(jnp.bfloat16)
    vib = v_i.astype(jnp.bfloat16)
    T = jnp.concatenate(
        [mbb[:, e * HD:(e + 1) * HD] * vib for e in range(E)] + [hmb],
        axis=1)

    w = weights(v_i, a1_i, a2_i)
    acc = jnp.dot(w, T, preferred_element_type=jnp.float32)
    rden = 1.0 / acc[:, E * HD:E * HD + E]
    rdenb = jnp.dot(rden, spr_ref[...],
                    preferred_element_type=jnp.float32)

    he_parts = []
    for e in range(E):
        sl = slice(e * HD, (e + 1) * HD)
        z = acc[:, sl] * rdenb[:, sl] + b_i + res_i
        after = jnp.where(z > 0, z, jnp.exp(z) - 1.0)
        allhf_ref[e, 0] = mb[:, sl] * after
        he_parts.append(jnp.max(jnp.where(mb[:, sl] > 0, after, -1e30),
                                axis=0, keepdims=True))
    he_ref[0] = jnp.concatenate(he_parts, axis=0)

    w_a = weights(v_a, a1_a, a2_a) * adj_ref[...]
    vab = jnp.concatenate(
        [v_a.astype(jnp.bfloat16),
         jnp.ones((v_a.shape[0], 1), jnp.bfloat16),
         jnp.zeros((v_a.shape[0], 7), jnp.bfloat16)], axis=1)
    acc_a = jnp.dot(w_a, vab, preferred_element_type=jnp.float32)
    rden_a = 1.0 / acc_a[:, HD:HD + 1]
    ones_row = jnp.ones((1, HD), jnp.float32)
    rdab = jnp.dot(rden_a, ones_row, preferred_element_type=jnp.float32)
    z = acc_a[:, 0:HD] * rdab + b_a + res_a
    ind_ref[0] = jnp.where(z > 0, z, jnp.exp(z) - 1.0)


def kernel(x, H, adj, nhid, W_i, a1_i, a2_i, b_i, Wres_i, bres_i,
           W_a, a1_a, a2_a, b_a, Wres_a, bres_a):
    B, S, F = x.shape
    HD = W_i.shape[1]
    E = H.shape[1]

    hm = (H != 0).astype(jnp.float32)
    adjb = adj.astype(jnp.bfloat16)
    zeros = jnp.zeros((HD,), jnp.float32)
    params = jnp.stack([a1_i[:, 0], a2_i[:, 0], b_i,
                        a1_a[:, 0], a2_a[:, 0], b_a, zeros, zeros])
    spread = (jnp.arange(E)[:, None] ==
              (jnp.arange(E * HD) // HD)[None, :]).astype(jnp.float32)

    allhf, he, ind = pl.pallas_call(
        functools.partial(_body, HD=HD, E=E),
        grid=(B,),
        in_specs=[
            pl.BlockSpec((1, S, F), lambda b: (b, 0, 0)),
            pl.BlockSpec((S, S), lambda b: (0, 0)),
            pl.BlockSpec((S, E), lambda b: (0, 0)),
            pl.BlockSpec((E, E * HD), lambda b: (0, 0)),
            pl.BlockSpec((8, HD), lambda b: (0, 0)),
            pl.BlockSpec((F, HD), lambda b: (0, 0)),
            pl.BlockSpec((F, HD), lambda b: (0, 0)),
            pl.BlockSpec((F, HD), lambda b: (0, 0)),
            pl.BlockSpec((F, HD), lambda b: (0, 0)),
            pl.BlockSpec((1, HD), lambda b: (0, 0)),
            pl.BlockSpec((1, HD), lambda b: (0, 0)),
        ],
        out_specs=[
            pl.BlockSpec((E, 1, S, HD), lambda b: (0, b, 0, 0)),
            pl.BlockSpec((1, E, HD), lambda b: (b, 0, 0)),
            pl.BlockSpec((1, S, HD), lambda b: (b, 0, 0)),
        ],
        out_shape=[
            jax.ShapeDtypeStruct((E, B, S, HD), jnp.float32),
            jax.ShapeDtypeStruct((B, E, HD), jnp.float32),
            jax.ShapeDtypeStruct((B, S, HD), jnp.float32),
        ],
        compiler_params=pltpu.CompilerParams(
            vmem_limit_bytes=100 * 1024 * 1024),
    )(x, adjb, hm, spread, params, W_i, Wres_i, W_a, Wres_a,
      bres_i[None, :], bres_a[None, :])

    return (allhf, he, ind)


# speedup vs baseline: 3.7100x; 1.0027x over previous
import functools

import jax
import jax.numpy as jnp
from jax.experimental import pallas as pl
from jax.experimental.pallas import tpu as pltpu


def _body(x_ref, adj_ref, hm_ref, spr_ref, par_ref, wi_ref, wri_ref,
          wa_ref, wra_ref, bri_ref, bra_ref, allhf_ref, he_ref, ind_ref,
          *, HD, E):
    xb = x_ref[0].astype(jnp.bfloat16)
    v_i = jnp.dot(xb, wi_ref[...].astype(jnp.bfloat16),
                  preferred_element_type=jnp.float32)
    v_a = jnp.dot(xb, wa_ref[...].astype(jnp.bfloat16),
                  preferred_element_type=jnp.float32)
    res_i = jnp.dot(xb, wri_ref[...].astype(jnp.bfloat16),
                    preferred_element_type=jnp.float32) + bri_ref[...]
    res_a = jnp.dot(xb, wra_ref[...].astype(jnp.bfloat16),
                    preferred_element_type=jnp.float32) + bra_ref[...]

    a1_i = par_ref[0:1, :]
    a2_i = par_ref[1:2, :]
    a1_a = par_ref[3:4, :]
    a2_a = par_ref[4:5, :]

    def weights(v, a1, a2):
        f2 = jax.lax.dot_general(a2, v, (((1,), (1,)), ((), ())),
                                 preferred_element_type=jnp.float32)
        m2 = jnp.max(f2)
        t = f2 - m2
        wp_c = jnp.exp(t).astype(jnp.bfloat16)
        wn_c = jnp.exp(0.2 * t).astype(jnp.bfloat16)
        nf2 = (-f2).astype(jnp.bfloat16)
        f1 = jax.lax.dot_general(v, a1, (((1,), (1,)), ((), ())),
                                 preferred_element_type=jnp.float32)
        u = f1 + m2
        m = jnp.where(u > 0, u, 0.2 * u)
        wp_r = jnp.exp(u - m).astype(jnp.bfloat16)
        wn_r = jnp.exp(0.2 * u - m).astype(jnp.bfloat16)
        f1b = f1.astype(jnp.bfloat16)
        return jnp.where(f1b > nf2, wp_r * wp_c, wn_r * wn_c)

    hmb = hm_ref[...].astype(jnp.bfloat16)
    sprb = spr_ref[...].astype(jnp.bfloat16)
    mb = jnp.dot(hmb, sprb,
                 preferred_element_type=jnp.float32)
    mbb = mb.astype(jnp.bfloat16)
    vib = v_i.astype(jnp.bfloat16)
    T = jnp.concatenate(
        [mbb[:, e * HD:(e + 1) * HD] * vib for e in range(E)] + [hmb],
        axis=1)

    w = weights(v_i, a1_i, a2_i)
    acc = jnp.dot(w, T, preferred_element_type=jnp.float32)
    rden = 1.0 / acc[:, E * HD:E * HD + E]
    rdenb = jnp.dot(rden, spr_ref[...],
                    preferred_element_type=jnp.float32)

    he_parts = []
    for e in range(E):
        sl = slice(e * HD, (e + 1) * HD)
        z = acc[:, sl] * rdenb[:, sl] + res_i
        after = jnp.where(z > 0, z, jnp.exp(z) - 1.0)
        masked = mb[:, sl] * after
        allhf_ref[e, 0] = masked
        he_parts.append(jnp.max(masked + 2.0 * mb[:, sl],
                                axis=0, keepdims=True))
    he_ref[0] = jnp.concatenate(he_parts, axis=0) - 2.0

    w_a = weights(v_a, a1_a, a2_a) * adj_ref[...]
    vab = jnp.concatenate(
        [v_a.astype(jnp.bfloat16),
         jnp.ones((v_a.shape[0], 1), jnp.bfloat16),
         jnp.zeros((v_a.shape[0], 7), jnp.bfloat16)], axis=1)
    acc_a = jnp.dot(w_a, vab, preferred_element_type=jnp.float32)
    rden_a = 1.0 / acc_a[:, HD:HD + 1]
    ones_row = jnp.ones((1, HD), jnp.float32)
    rdab = jnp.dot(rden_a, ones_row, preferred_element_type=jnp.float32)
    z = acc_a[:, 0:HD] * rdab + res_a
    ind_ref[0] = jnp.where(z > 0, z, jnp.exp(z) - 1.0)


def kernel(x, H, adj, nhid, W_i, a1_i, a2_i, b_i, Wres_i, bres_i,
           W_a, a1_a, a2_a, b_a, Wres_a, bres_a):
    B, S, F = x.shape
    HD = W_i.shape[1]
    E = H.shape[1]

    hm = (H != 0).astype(jnp.float32)
    adjb = adj.astype(jnp.bfloat16)
    zeros = jnp.zeros((HD,), jnp.float32)
    params = jnp.stack([a1_i[:, 0], a2_i[:, 0], zeros,
                        a1_a[:, 0], a2_a[:, 0], zeros, zeros, zeros])
    bri = (bres_i + b_i)[None, :]
    bra = (bres_a + b_a)[None, :]
    spread = (jnp.arange(E)[:, None] ==
              (jnp.arange(E * HD) // HD)[None, :]).astype(jnp.float32)

    allhf, he, ind = pl.pallas_call(
        functools.partial(_body, HD=HD, E=E),
        grid=(B,),
        in_specs=[
            pl.BlockSpec((1, S, F), lambda b: (b, 0, 0)),
            pl.BlockSpec((S, S), lambda b: (0, 0)),
            pl.BlockSpec((S, E), lambda b: (0, 0)),
            pl.BlockSpec((E, E * HD), lambda b: (0, 0)),
            pl.BlockSpec((8, HD), lambda b: (0, 0)),
            pl.BlockSpec((F, HD), lambda b: (0, 0)),
            pl.BlockSpec((F, HD), lambda b: (0, 0)),
            pl.BlockSpec((F, HD), lambda b: (0, 0)),
            pl.BlockSpec((F, HD), lambda b: (0, 0)),
            pl.BlockSpec((1, HD), lambda b: (0, 0)),
            pl.BlockSpec((1, HD), lambda b: (0, 0)),
        ],
        out_specs=[
            pl.BlockSpec((E, 1, S, HD), lambda b: (0, b, 0, 0)),
            pl.BlockSpec((1, E, HD), lambda b: (b, 0, 0)),
            pl.BlockSpec((1, S, HD), lambda b: (b, 0, 0)),
        ],
        out_shape=[
            jax.ShapeDtypeStruct((E, B, S, HD), jnp.float32),
            jax.ShapeDtypeStruct((B, E, HD), jnp.float32),
            jax.ShapeDtypeStruct((B, S, HD), jnp.float32),
        ],
        compiler_params=pltpu.CompilerParams(
            vmem_limit_bytes=100 * 1024 * 1024),
    )(x, adjb, hm, spread, params, W_i, Wres_i, W_a, Wres_a, bri, bra)

    return (allhf, he, ind)
